# baseline (speedup 1.0000x reference)
import functools

import jax
import jax.numpy as jnp
from jax.experimental import pallas as pl

_B, _N = 4096, 50
_D1, _D2 = 32, 32
_H, _O, _R = 64, 32, 8


def _leaky(x):
    return jnp.where(x > 0, x, jnp.asarray(0.01, x.dtype) * x)


def _ui_kernel(u_ref, i_ref, Wu1_ref, bu1_ref, Wu2_ref, bu2_ref, ue_ref):
    ui_in = jnp.concatenate([u_ref[...], i_ref[...]], axis=-1)
    hu = _leaky(jnp.dot(ui_in, Wu1_ref[...],
                        preferred_element_type=jnp.float32) + bu1_ref[...])
    ue_ref[...] = _leaky(jnp.dot(hu, Wu2_ref[...],
                                 preferred_element_type=jnp.float32)
                         + bu2_ref[...])


def _fused_kernel(ue_ref, a_ref, o_ref, s_ref,
                  W1a_ref, W2a_ref,
                  E1_ref, E2_ref,
                  out_ref, *, bb, n):
    rows = bb * n
    ones3 = jnp.ones((bb, n, 1), dtype=jnp.bfloat16)
    ao3 = jnp.concatenate([a_ref[...].astype(jnp.bfloat16),
                           o_ref[...].astype(jnp.bfloat16), ones3], axis=-1)
    ao = ao3.reshape(rows, 2 * _D1 + 1)

    rid = jax.lax.broadcasted_iota(jnp.int32, (bb, n, _R), 2)
    m = (s_ref[...][:, :, None] == rid).astype(jnp.bfloat16).reshape(rows, _R)

    z1 = jnp.dot(ao, W1a_ref[...],
                 preferred_element_type=jnp.float32).astype(jnp.bfloat16)
    M1 = jnp.dot(m, E1_ref[...],
                 preferred_element_type=jnp.float32).astype(jnp.bfloat16)
    z1m = z1 * M1
    h_pre = z1m[:, :_H]
    for r in range(1, _R):
        h_pre = h_pre + z1m[:, r * _H:(r + 1) * _H]
    h_sel = _leaky(h_pre)

    ones2 = jnp.ones((rows, 1), dtype=jnp.bfloat16)
    ha = jnp.concatenate([h_sel, ones2], axis=-1)
    z2 = jnp.dot(ha, W2a_ref[...],
                 preferred_element_type=jnp.float32).astype(jnp.bfloat16)
    M2 = jnp.dot(m, E2_ref[...],
                 preferred_element_type=jnp.float32).astype(jnp.bfloat16)
    z2m = z2 * M2
    o_pre = z2m[:, :_O]
    for r in range(1, _R):
        o_pre = o_pre + z2m[:, r * _O:(r + 1) * _O]
    o_sel = _leaky(o_pre)

    o3 = o_sel.reshape(bb, n, _O).astype(jnp.float32)
    ue = ue_ref[...]
    out_ref[...] = jnp.sum(o3 * ue[:, None, :], axis=-1)


@jax.jit
def kernel(u_emb, i_emb, a_emb, o_emb, s,
           W_ui1, b_ui1, W_ui2, b_ui2, W_ao1, b_ao1, W_ao2, b_ao2):
    BB = 128
    grid = (_B // BB,)

    W1c = jnp.transpose(W_ao1, (1, 0, 2)).reshape(2 * _D1, _R * _H)
    W1a = jnp.concatenate([W1c, b_ao1.reshape(1, _R * _H)], axis=0)
    W1a = W1a.astype(jnp.bfloat16)
    W2c = jnp.transpose(W_ao2, (1, 0, 2)).reshape(_H, _R * _O)
    W2a = jnp.concatenate([W2c, b_ao2.reshape(1, _R * _O)], axis=0)
    W2a = W2a.astype(jnp.bfloat16)
    bu1 = b_ui1.reshape(1, _H)
    bu2 = b_ui2.reshape(1, _O)
    E1 = jnp.repeat(jnp.eye(_R, dtype=jnp.bfloat16), _H, axis=1)
    E2 = jnp.repeat(jnp.eye(_R, dtype=jnp.bfloat16), _O, axis=1)

    ue = pl.pallas_call(
        _ui_kernel,
        out_shape=jax.ShapeDtypeStruct((_B, _O), jnp.float32),
    )(u_emb, i_emb, W_ui1, bu1, W_ui2, bu2)

    full = lambda *shape: pl.BlockSpec(shape, lambda i: (0,) * len(shape))
    out = pl.pallas_call(
        functools.partial(_fused_kernel, bb=BB, n=_N),
        grid=grid,
        in_specs=[
            pl.BlockSpec((BB, _O), lambda i: (i, 0)),
            pl.BlockSpec((BB, _N, _D1), lambda i: (i, 0, 0)),
            pl.BlockSpec((BB, _N, _D1), lambda i: (i, 0, 0)),
            pl.BlockSpec((BB, _N), lambda i: (i, 0)),
            full(2 * _D1 + 1, _R * _H),
            full(_H + 1, _R * _O),
            full(_R, _R * _H),
            full(_R, _R * _O),
        ],
        out_specs=pl.BlockSpec((BB, _N), lambda i: (i, 0)),
        out_shape=jax.ShapeDtypeStruct((_B, _N), jnp.float32),
    )(ue, a_emb, o_emb, s, W1a, W2a, E1, E2)
    return out


# speedup vs baseline: 1.0215x; 1.0215x over previous
import functools

import jax
import jax.numpy as jnp
from jax.experimental import pallas as pl

_B, _N = 4096, 50
_D1, _D2 = 32, 32
_H, _O, _R = 64, 32, 8


def _leaky(x):
    return jnp.where(x > 0, x, jnp.asarray(0.01, x.dtype) * x)


def _ui_kernel(u_ref, i_ref, Wu1_ref, bu1_ref, Wu2_ref, bu2_ref, ue_ref):
    ui_in = jnp.concatenate([u_ref[...], i_ref[...]], axis=-1)
    hu = _leaky(jnp.dot(ui_in, Wu1_ref[...],
                        preferred_element_type=jnp.float32) + bu1_ref[...])
    ue_ref[...] = _leaky(jnp.dot(hu, Wu2_ref[...],
                                 preferred_element_type=jnp.float32)
                         + bu2_ref[...])


def _fused_kernel(ue_ref, a_ref, o_ref, s_ref,
                  W1a_ref, W2a_ref,
                  E1_ref, E2_ref,
                  out_ref, *, bb, n):
    rows = bb * n
    ones3 = jnp.ones((bb, n, 1), dtype=jnp.bfloat16)
    ao3 = jnp.concatenate([a_ref[...].astype(jnp.bfloat16),
                           o_ref[...].astype(jnp.bfloat16), ones3], axis=-1)
    ao = ao3.reshape(rows, 2 * _D1 + 1)

    rid = jax.lax.broadcasted_iota(jnp.int32, (bb, n, _R), 2)
    m = (s_ref[...][:, :, None] == rid).astype(jnp.bfloat16).reshape(rows, _R)

    z1 = jnp.dot(ao, W1a_ref[...],
                 preferred_element_type=jnp.float32).astype(jnp.bfloat16)
    M1 = jnp.dot(m, E1_ref[...],
                 preferred_element_type=jnp.float32).astype(jnp.bfloat16)
    z1m = z1 * M1
    h_pre = z1m[:, :_H]
    for r in range(1, _R):
        h_pre = h_pre + z1m[:, r * _H:(r + 1) * _H]
    h_sel = _leaky(h_pre)

    ones2 = jnp.ones((rows, 1), dtype=jnp.bfloat16)
    ha = jnp.concatenate([h_sel, ones2], axis=-1)
    z2 = jnp.dot(ha, W2a_ref[...],
                 preferred_element_type=jnp.float32).astype(jnp.bfloat16)
    M2 = jnp.dot(m, E2_ref[...],
                 preferred_element_type=jnp.float32).astype(jnp.bfloat16)
    z2m = z2 * M2
    o_pre = z2m[:, :_O]
    for r in range(1, _R):
        o_pre = o_pre + z2m[:, r * _O:(r + 1) * _O]
    o_sel = _leaky(o_pre)

    o3 = o_sel.reshape(bb, n, _O).astype(jnp.float32)
    ue = ue_ref[...]
    out_ref[...] = jnp.sum(o3 * ue[:, None, :], axis=-1)


@jax.jit
def kernel(u_emb, i_emb, a_emb, o_emb, s,
           W_ui1, b_ui1, W_ui2, b_ui2, W_ao1, b_ao1, W_ao2, b_ao2):
    BB = 64
    grid = (_B // BB,)

    W1c = jnp.transpose(W_ao1, (1, 0, 2)).reshape(2 * _D1, _R * _H)
    W1a = jnp.concatenate([W1c, b_ao1.reshape(1, _R * _H)], axis=0)
    W1a = W1a.astype(jnp.bfloat16)
    W2c = jnp.transpose(W_ao2, (1, 0, 2)).reshape(_H, _R * _O)
    W2a = jnp.concatenate([W2c, b_ao2.reshape(1, _R * _O)], axis=0)
    W2a = W2a.astype(jnp.bfloat16)
    bu1 = b_ui1.reshape(1, _H)
    bu2 = b_ui2.reshape(1, _O)
    E1 = jnp.repeat(jnp.eye(_R, dtype=jnp.bfloat16), _H, axis=1)
    E2 = jnp.repeat(jnp.eye(_R, dtype=jnp.bfloat16), _O, axis=1)

    ue = pl.pallas_call(
        _ui_kernel,
        out_shape=jax.ShapeDtypeStruct((_B, _O), jnp.float32),
    )(u_emb, i_emb, W_ui1, bu1, W_ui2, bu2)

    full = lambda *shape: pl.BlockSpec(shape, lambda i: (0,) * len(shape))
    out = pl.pallas_call(
        functools.partial(_fused_kernel, bb=BB, n=_N),
        grid=grid,
        in_specs=[
            pl.BlockSpec((BB, _O), lambda i: (i, 0)),
            pl.BlockSpec((BB, _N, _D1), lambda i: (i, 0, 0)),
            pl.BlockSpec((BB, _N, _D1), lambda i: (i, 0, 0)),
            pl.BlockSpec((BB, _N), lambda i: (i, 0)),
            full(2 * _D1 + 1, _R * _H),
            full(_H + 1, _R * _O),
            full(_R, _R * _H),
            full(_R, _R * _O),
        ],
        out_specs=pl.BlockSpec((BB, _N), lambda i: (i, 0)),
        out_shape=jax.ShapeDtypeStruct((_B, _N), jnp.float32),
    )(ue, a_emb, o_emb, s, W1a, W2a, E1, E2)
    return out
